# Initial kernel scaffold; baseline (speedup 1.0000x reference)
import jax
import jax.numpy as jnp
from jax.experimental import pallas as pl


def kernel(sentence, table):
    raise NotImplementedError("write your pallas kernel here")


# speedup vs baseline: 1.8523x; 1.8523x over previous
import functools

import jax
import jax.numpy as jnp
from jax import lax
from jax.experimental import pallas as pl
from jax.experimental.pallas import tpu as pltpu
from jax.experimental.pallas import tpu_sc as plsc

_BATCH = 16384
_SEQ = 50
_D = 64
_B = _BATCH * _SEQ
_NC = 2
_NS = 16
_NW = _NC * _NS
_BPW = _B // _NW
_CH = 512
_NCHUNK = _BPW // _CH
_NBUF = 2
_ROUNDS = _NCHUNK // _NBUF


@functools.partial(
    pl.kernel,
    out_type=jax.ShapeDtypeStruct((_B, _D), jnp.float32),
    mesh=plsc.VectorSubcoreMesh(core_axis_name="c", subcore_axis_name="s"),
    compiler_params=pltpu.CompilerParams(use_tc_tiling_on_sc=False),
    scratch_types=[
        pltpu.VMEM((_CH,), jnp.int32),
        pltpu.VMEM((_CH,), jnp.int32),
        pltpu.VMEM((_CH, _D), jnp.float32),
        pltpu.VMEM((_CH, _D), jnp.float32),
        pltpu.SemaphoreType.DMA,
        pltpu.SemaphoreType.DMA,
    ],
)
def _embed_gather(idx_hbm, table_hbm, out_hbm,
                  idx0, idx1, rows0, rows1, sem0, sem1):
    idx_v = [idx0, idx1]
    rows_v = [rows0, rows1]
    sems = [sem0, sem1]

    wid = lax.axis_index("s") * _NC + lax.axis_index("c")
    base = wid * _BPW

    def _start(b, chunk):
        off = base + chunk * _CH
        pltpu.sync_copy(idx_hbm.at[pl.ds(off, _CH)], idx_v[b])
        pltpu.make_async_copy(
            table_hbm.at[idx_v[b]], rows_v[b], sems[b]).start()

    def _finish(b, chunk):
        off = base + chunk * _CH
        pltpu.make_async_copy(
            table_hbm.at[idx_v[b]], rows_v[b], sems[b]).wait()
        pltpu.sync_copy(rows_v[b], out_hbm.at[pl.ds(off, _CH)])

    for b in range(_NBUF):
        _start(b, b)

    def _round(t, carry):
        for b in range(_NBUF):
            c = t * _NBUF + b
            _finish(b, c)
            _start(b, c + _NBUF)
        return carry

    lax.fori_loop(0, _ROUNDS - 1, _round, 0)
    for b in range(_NBUF):
        _finish(b, (_ROUNDS - 1) * _NBUF + b)


def kernel(sentence, table):
    idx = sentence.reshape(_B).astype(jnp.int32)
    out = _embed_gather(idx, table)
    return out.reshape(_BATCH, _SEQ, _D)
